# Initial kernel scaffold; baseline (speedup 1.0000x reference)
import jax
import jax.numpy as jnp
from jax.experimental import pallas as pl


def kernel(x, l0_wq1, l0_bq1, l0_wq2, l0_bq2, l0_ws1, l0_bs1, l0_ws2, l0_bs2, l0_wqc, l0_bqc, l0_wsc, l0_bsc, l0_gn_gamma, l0_gn_beta, l1_wq1, l1_bq1, l1_wq2, l1_bq2, l1_ws1, l1_bs1, l1_ws2, l1_bs2, l1_wqc, l1_bqc, l1_wsc, l1_bsc, l1_gn_gamma, l1_gn_beta):
    raise NotImplementedError("write your pallas kernel here")


# speedup vs baseline: 1.0217x; 1.0217x over previous
import functools
import jax
import jax.numpy as jnp
from jax import lax
from jax.experimental import pallas as pl
from jax.experimental.pallas import tpu as pltpu


def _branch_body(x_ref, w1_ref, b1_ref, wk_ref, bf_ref, o_ref, *, N):
    X = x_ref[0, 0]
    Y1 = jnp.dot(X, w1_ref[0], preferred_element_type=jnp.float32) + b1_ref[0]
    Yb = Y1.astype(jnp.bfloat16)
    rows, le = Yb.shape
    z = jnp.zeros((N, le), jnp.bfloat16)
    up = jnp.concatenate([z, Yb[:rows - N]], axis=0)
    dn = jnp.concatenate([Yb[N:], z], axis=0)
    acc = jnp.dot(up, wk_ref[0, 0], preferred_element_type=jnp.float32)
    acc = acc + jnp.dot(Yb, wk_ref[0, 1], preferred_element_type=jnp.float32)
    acc = acc + jnp.dot(dn, wk_ref[0, 2], preferred_element_type=jnp.float32)
    o_ref[0, 0] = acc + bf_ref[0]


def _branch_pair(x_stack, w1_s, b1_s, wk_s, bf_s, *, N):
    _, B, DN, LF = x_stack.shape
    LE = w1_s.shape[2]
    KH = wk_s.shape[1]
    CF = wk_s.shape[3]
    body = functools.partial(_branch_body, N=N)
    return pl.pallas_call(
        body,
        out_shape=jax.ShapeDtypeStruct((2, B, DN, CF), jnp.float32),
        grid=(2, B),
        in_specs=[
            pl.BlockSpec((1, 1, DN, LF), lambda r, b: (r, b, 0, 0)),
            pl.BlockSpec((1, LF, LE), lambda r, b: (r, 0, 0)),
            pl.BlockSpec((1, 1, LE), lambda r, b: (r, 0, 0)),
            pl.BlockSpec((1, KH, LE, CF), lambda r, b: (r, 0, 0, 0)),
            pl.BlockSpec((1, 1, CF), lambda r, b: (r, 0, 0)),
        ],
        out_specs=pl.BlockSpec((1, 1, DN, CF), lambda r, b: (r, b, 0, 0)),
        compiler_params=pltpu.CompilerParams(
            dimension_semantics=("parallel", "parallel"),
            vmem_limit_bytes=64 * 1024 * 1024),
    )(x_stack, w1_s, b1_s, wk_s, bf_s)


def _gn_body(q_ref, s_ref, g_ref, b_ref, o_ref, *, groups, eps):
    x = q_ref[0] + s_ref[0]
    C, S = x.shape
    Cg = C // groups
    cnt = float(Cg * S)
    mus, invs = [], []
    for g in range(groups):
        xg = x[g * Cg:(g + 1) * Cg, :]
        mu = jnp.sum(xg) / cnt
        var = jnp.sum((xg - mu) * (xg - mu)) / cnt
        inv = lax.rsqrt(var + eps)
        mus.append(jnp.full((Cg, 1), 0.0, jnp.float32) + mu)
        invs.append(jnp.full((Cg, 1), 0.0, jnp.float32) + inv)
    mu_c = jnp.concatenate(mus, axis=0)
    inv_c = jnp.concatenate(invs, axis=0)
    o_ref[0] = jnp.maximum((x - mu_c) * (inv_c * g_ref[0]) + b_ref[0], 0.0)


def _add_gn_relu(q3, s3, gamma, beta, *, groups, eps=1e-5):
    B, C, S = q3.shape
    body = functools.partial(_gn_body, groups=groups, eps=eps)
    return pl.pallas_call(
        body,
        out_shape=jax.ShapeDtypeStruct((B, C, S), jnp.float32),
        grid=(B,),
        in_specs=[
            pl.BlockSpec((1, C, S), lambda b: (b, 0, 0)),
            pl.BlockSpec((1, C, S), lambda b: (b, 0, 0)),
            pl.BlockSpec((1, C, 1), lambda b: (0, 0, 0)),
            pl.BlockSpec((1, C, 1), lambda b: (0, 0, 0)),
        ],
        out_specs=pl.BlockSpec((1, C, S), lambda b: (b, 0, 0)),
        compiler_params=pltpu.CompilerParams(
            dimension_semantics=("parallel",),
            vmem_limit_bytes=64 * 1024 * 1024),
    )(q3, s3, gamma.reshape(1, C, 1), beta.reshape(1, C, 1))


def _prep_weights(w1, b1, wc, bc, w2, b2):
    E, F = w1.shape
    C, L, KH, KW = wc.shape
    pad_w = (KW - 1) // 2
    w1blk = jnp.kron(jnp.eye(L, dtype=w1.dtype), w1.T)
    b1row = jnp.tile(b1, (L,)).reshape(1, L * E)
    shifted = []
    for ke in range(KW):
        de = ke - pad_w
        if de > 0:
            s = jnp.concatenate(
                [jnp.zeros((F, de), w2.dtype), w2[:, :-de]], axis=1)
        elif de < 0:
            s = jnp.concatenate(
                [w2[:, -de:], jnp.zeros((F, -de), w2.dtype)], axis=1)
        else:
            s = w2
        shifted.append(s)
    w2s = jnp.stack(shifted, axis=0)
    wk = jnp.einsum('olab,bfe->aleof', wc, w2s).reshape(KH, L * E, C * F)
    brow = (jnp.outer(bc, jnp.sum(w2, axis=1)) + b2[None, :]).reshape(1, C * F)
    return w1blk, b1row, wk, brow


def _layer(x, p, groups):
    B, L, D, Hq, Wq, Hs, Ws = x.shape
    F = Hq * Wq
    N = Hs * Ws
    C = p['wqc'].shape[0]

    xq = x.transpose(0, 2, 5, 6, 1, 3, 4).reshape(B, D * N, L * F)
    xs = x.transpose(0, 2, 3, 4, 1, 5, 6).reshape(B, D * F, L * F)
    x_stack = jnp.stack([xq, xs], axis=0).astype(jnp.bfloat16)

    wq = _prep_weights(p['wq1'], p['bq1'], p['wqc'], p['bqc'],
                       p['wq2'], p['bq2'])
    ws = _prep_weights(p['ws1'], p['bs1'], p['wsc'], p['bsc'],
                       p['ws2'], p['bs2'])
    w1_s, b1_s, wk_s, bf_s = [jnp.stack([a, b], axis=0)
                              for a, b in zip(wq, ws)]
    w1_s = w1_s.astype(jnp.bfloat16)
    wk_s = wk_s.astype(jnp.bfloat16)

    y = _branch_pair(x_stack, w1_s, b1_s, wk_s, bf_s, N=N)

    q7 = y[0].reshape(B, D, Hs, Ws, C, Hq, Wq).transpose(0, 4, 1, 5, 6, 2, 3)
    s7 = y[1].reshape(B, D, Hq, Wq, C, Hs, Ws).transpose(0, 4, 1, 2, 3, 5, 6)
    S = D * Hq * Wq * Hs * Ws
    out3 = _add_gn_relu(q7.reshape(B, C, S), s7.reshape(B, C, S),
                        p['gn_gamma'], p['gn_beta'], groups=groups)
    return out3.reshape(B, C, D, Hq, Wq, Hs, Ws)


def kernel(x,
           l0_wq1, l0_bq1, l0_wq2, l0_bq2, l0_ws1, l0_bs1, l0_ws2, l0_bs2,
           l0_wqc, l0_bqc, l0_wsc, l0_bsc, l0_gn_gamma, l0_gn_beta,
           l1_wq1, l1_bq1, l1_wq2, l1_bq2, l1_ws1, l1_bs1, l1_ws2, l1_bs2,
           l1_wqc, l1_bqc, l1_wsc, l1_bsc, l1_gn_gamma, l1_gn_beta):
    p0 = dict(wq1=l0_wq1, bq1=l0_bq1, wq2=l0_wq2, bq2=l0_bq2,
              ws1=l0_ws1, bs1=l0_bs1, ws2=l0_ws2, bs2=l0_bs2,
              wqc=l0_wqc, bqc=l0_bqc, wsc=l0_wsc, bsc=l0_bsc,
              gn_gamma=l0_gn_gamma, gn_beta=l0_gn_beta)
    p1 = dict(wq1=l1_wq1, bq1=l1_bq1, wq2=l1_wq2, bq2=l1_bq2,
              ws1=l1_ws1, bs1=l1_bs1, ws2=l1_ws2, bs2=l1_bs2,
              wqc=l1_wqc, bqc=l1_bqc, wsc=l1_wsc, bsc=l1_bsc,
              gn_gamma=l1_gn_gamma, gn_beta=l1_gn_beta)
    residuals = [x]
    y0 = _layer(x, p0, groups=4)
    residuals.append(y0)
    y1 = _layer(y0, p1, groups=4)
    return y1, residuals


# speedup vs baseline: 3.4344x; 3.3613x over previous
import functools
import jax
import jax.numpy as jnp
from jax import lax
from jax.experimental import pallas as pl
from jax.experimental.pallas import tpu as pltpu


def _blockT(y, R, W):
    rows, cols = y.shape
    CB = cols // W
    y3 = y.reshape(R, W, cols)
    yt = jnp.swapaxes(y3, 1, 2)
    parts = [yt[:, c * W:(c + 1) * W, :].reshape(rows, W)
             for c in range(CB)]
    if CB == 1:
        return parts[0]
    return jnp.concatenate(parts, axis=1)


def _taps(Yb, wk_ref, br, bf, W):
    rows, le = Yb.shape
    z = jnp.zeros((W, le), jnp.bfloat16)
    up = jnp.concatenate([z, Yb[:rows - W]], axis=0)
    dn = jnp.concatenate([Yb[W:], z], axis=0)
    acc = jnp.dot(up, wk_ref[br, 0], preferred_element_type=jnp.float32)
    acc = acc + jnp.dot(Yb, wk_ref[br, 1], preferred_element_type=jnp.float32)
    acc = acc + jnp.dot(dn, wk_ref[br, 2], preferred_element_type=jnp.float32)
    return acc + bf


def _gn_rows(z, groups, eps, grow, brow):
    R, cf = z.shape
    gw = cf // groups
    cnt = float(R * gw)
    mus, invs = [], []
    for g in range(groups):
        zg = z[:, g * gw:(g + 1) * gw]
        s1 = jnp.sum(zg, axis=0, keepdims=True)
        s2 = jnp.sum(zg * zg, axis=0, keepdims=True)
        m1 = jnp.sum(s1, axis=1, keepdims=True) / cnt
        m2 = jnp.sum(s2, axis=1, keepdims=True) / cnt
        inv = lax.rsqrt(m2 - m1 * m1 + eps)
        mus.append(jnp.broadcast_to(m1, (1, gw)))
        invs.append(jnp.broadcast_to(inv, (1, gw)))
    mu = jnp.concatenate(mus, axis=1)
    inv = jnp.concatenate(invs, axis=1)
    scale = inv * grow
    shift = brow - mu * scale
    return scale, shift


def _concat_c_rows(a, C):
    rows, cf = a.shape
    W = cf // C
    return jnp.concatenate([a[:, c * W:(c + 1) * W] for c in range(C)], axis=0)


def _net_body(x_ref,
              w1_0, b1_0, wk_0, bf_0, gr_0, br_0,
              w1_1, b1_1, wk_1, bf_1, gr_1, br_1,
              res_ref, out_ref, *, L, D, groups, eps):
    xb = x_ref[0]
    W = xb.shape[2]

    x4 = xb.reshape(L * D, W, W)
    xt = jnp.swapaxes(x4, 1, 2)
    xq = jnp.concatenate(
        [xt[l * D:(l + 1) * D].reshape(D * W, W) for l in range(L)],
        axis=1)
    xs = jnp.concatenate([xb[l] for l in range(L)], axis=1)

    y1q = jnp.dot(xq, w1_0[0], preferred_element_type=jnp.float32) + b1_0[0]
    y1s = jnp.dot(xs, w1_0[1], preferred_element_type=jnp.float32) + b1_0[1]
    yq = _taps(y1q.astype(jnp.bfloat16), wk_0, 0, bf_0[0], W)
    ys = _taps(y1s.astype(jnp.bfloat16), wk_0, 1, bf_0[1], W)

    z_q = yq + _blockT(ys, D, W)
    z_s = _blockT(yq, D, W) + ys
    scale, shift = _gn_rows(z_q, groups, eps, gr_0[0], br_0[0])
    a_q = jnp.maximum(z_q * scale + shift, 0.0)
    a_s = jnp.maximum(z_s * scale + shift, 0.0)

    C = gr_0.shape[1] // W
    res_ref[0] = _concat_c_rows(a_s, C)

    y1q = jnp.dot(a_q.astype(jnp.bfloat16), w1_1[0],
                  preferred_element_type=jnp.float32) + b1_1[0]
    y1s = jnp.dot(a_s.astype(jnp.bfloat16), w1_1[1],
                  preferred_element_type=jnp.float32) + b1_1[1]
    yq = _taps(y1q.astype(jnp.bfloat16), wk_1, 0, bf_1[0], W)
    ys = _taps(y1s.astype(jnp.bfloat16), wk_1, 1, bf_1[1], W)

    z_s = _blockT(yq, D, W) + ys
    scale, shift = _gn_rows(z_s, groups, eps, gr_1[0], br_1[0])
    out = jnp.maximum(z_s * scale + shift, 0.0)
    out_ref[0] = _concat_c_rows(out, C)


def _prep_weights(w1, b1, wc, bc, w2, b2):
    E, F = w1.shape
    C, L, KH, KW = wc.shape
    pad_w = (KW - 1) // 2
    w1blk = jnp.kron(jnp.eye(L, dtype=w1.dtype), w1.T)
    b1row = jnp.tile(b1, (L,)).reshape(1, L * E)
    shifted = []
    for ke in range(KW):
        de = ke - pad_w
        if de > 0:
            s = jnp.concatenate(
                [jnp.zeros((F, de), w2.dtype), w2[:, :-de]], axis=1)
        elif de < 0:
            s = jnp.concatenate(
                [w2[:, -de:], jnp.zeros((F, -de), w2.dtype)], axis=1)
        else:
            s = w2
        shifted.append(s)
    w2s = jnp.stack(shifted, axis=0)
    wk = jnp.einsum('olab,bfe->aleof', wc, w2s).reshape(KH, L * E, C * F)
    brow = (jnp.outer(bc, jnp.sum(w2, axis=1)) + b2[None, :]).reshape(1, C * F)
    return w1blk, b1row, wk, brow


def _layer_weights(p):
    wq = _prep_weights(p['wq1'], p['bq1'], p['wqc'], p['bqc'],
                       p['wq2'], p['bq2'])
    ws = _prep_weights(p['ws1'], p['bs1'], p['wsc'], p['bsc'],
                       p['ws2'], p['bs2'])
    w1 = jnp.stack([wq[0], ws[0]]).astype(jnp.bfloat16)
    b1 = jnp.stack([wq[1], ws[1]])
    wk = jnp.stack([wq[2], ws[2]]).astype(jnp.bfloat16)
    bf = jnp.stack([wq[3], ws[3]])
    C = p['wqc'].shape[0]
    F = p['wq1'].shape[1]
    gr = jnp.repeat(p['gn_gamma'], F).reshape(1, C * F)
    br = jnp.repeat(p['gn_beta'], F).reshape(1, C * F)
    return w1, b1, wk, bf, gr, br


def _run_net(x, p0, p1, groups, eps=1e-5):
    B, L, D, Hq, Wq, Hs, Ws = x.shape
    F = Hq * Wq
    N = Hs * Ws
    C = p0['wqc'].shape[0]
    x3 = x.reshape(B, L, D * F, N).astype(jnp.bfloat16)

    wa = _layer_weights(p0)
    wb = _layer_weights(p1)
    LF0, LE0 = wa[0].shape[1], wa[0].shape[2]
    LF1, LE1 = wb[0].shape[1], wb[0].shape[2]
    CF = wa[2].shape[3]
    KH = wa[2].shape[1]

    body = functools.partial(_net_body, L=L, D=D, groups=groups, eps=eps)
    cmap = lambda b: (0, 0, 0)
    cmap4 = lambda b: (0, 0, 0, 0)
    res, out = pl.pallas_call(
        body,
        out_shape=(
            jax.ShapeDtypeStruct((B, C * D * F, N), jnp.float32),
            jax.ShapeDtypeStruct((B, C * D * F, N), jnp.float32),
        ),
        grid=(B,),
        in_specs=[
            pl.BlockSpec((1, L, D * F, N), lambda b: (b, 0, 0, 0)),
            pl.BlockSpec((2, LF0, LE0), cmap),
            pl.BlockSpec((2, 1, LE0), cmap),
            pl.BlockSpec((2, KH, LE0, CF), cmap4),
            pl.BlockSpec((2, 1, CF), cmap),
            pl.BlockSpec((1, CF), lambda b: (0, 0)),
            pl.BlockSpec((1, CF), lambda b: (0, 0)),
            pl.BlockSpec((2, LF1, LE1), cmap),
            pl.BlockSpec((2, 1, LE1), cmap),
            pl.BlockSpec((2, KH, LE1, CF), cmap4),
            pl.BlockSpec((2, 1, CF), cmap),
            pl.BlockSpec((1, CF), lambda b: (0, 0)),
            pl.BlockSpec((1, CF), lambda b: (0, 0)),
        ],
        out_specs=(
            pl.BlockSpec((1, C * D * F, N), lambda b: (b, 0, 0)),
            pl.BlockSpec((1, C * D * F, N), lambda b: (b, 0, 0)),
        ),
        compiler_params=pltpu.CompilerParams(
            dimension_semantics=("parallel",),
            vmem_limit_bytes=100 * 1024 * 1024),
    )(x3, *wa, *wb)
    shape7 = (B, C, D, Hq, Wq, Hs, Ws)
    return res.reshape(shape7), out.reshape(shape7)


def kernel(x,
           l0_wq1, l0_bq1, l0_wq2, l0_bq2, l0_ws1, l0_bs1, l0_ws2, l0_bs2,
           l0_wqc, l0_bqc, l0_wsc, l0_bsc, l0_gn_gamma, l0_gn_beta,
           l1_wq1, l1_bq1, l1_wq2, l1_bq2, l1_ws1, l1_bs1, l1_ws2, l1_bs2,
           l1_wqc, l1_bqc, l1_wsc, l1_bsc, l1_gn_gamma, l1_gn_beta):
    p0 = dict(wq1=l0_wq1, bq1=l0_bq1, wq2=l0_wq2, bq2=l0_bq2,
              ws1=l0_ws1, bs1=l0_bs1, ws2=l0_ws2, bs2=l0_bs2,
              wqc=l0_wqc, bqc=l0_bqc, wsc=l0_wsc, bsc=l0_bsc,
              gn_gamma=l0_gn_gamma, gn_beta=l0_gn_beta)
    p1 = dict(wq1=l1_wq1, bq1=l1_bq1, wq2=l1_wq2, bq2=l1_bq2,
              ws1=l1_ws1, bs1=l1_bs1, ws2=l1_ws2, bs2=l1_bs2,
              wqc=l1_wqc, bqc=l1_bqc, wsc=l1_wsc, bsc=l1_bsc,
              gn_gamma=l1_gn_gamma, gn_beta=l1_gn_beta)
    y0, y1 = _run_net(x, p0, p1, groups=4)
    return y1, [x, y0]
